# baseline (speedup 1.0000x reference)
import functools

import jax
import jax.numpy as jnp
import numpy as np
from jax import lax
from jax.experimental import pallas as pl
from jax.experimental.pallas import tpu as pltpu

_EPS = 1e-6
_TINY = 1.1754943508222875e-38
_KS1 = 42
_KS2 = 0x1BD11BF0
_ROT = ((13, 15, 26, 6), (17, 29, 16, 24))
_LANES = 128
_IMGS = 4


def _threefry_bits(n):
    ks = (0, _KS1, _KS2)
    x0 = jnp.zeros_like(n)
    x1 = n + jnp.uint32(_KS1)
    for i in range(5):
        for r in _ROT[i % 2]:
            x0 = x0 + x1
            x1 = (x1 << r) | (x1 >> (32 - r))
            x1 = x1 ^ x0
        x0 = x0 + jnp.uint32(ks[(i + 1) % 3])
        x1 = x1 + jnp.uint32(ks[(i + 2) % 3] + (i + 1))
    return x0 ^ x1


def _gumbel(n):
    bits = _threefry_bits(n)
    mant = (bits >> 9) | jnp.uint32(0x3F800000)
    u0 = pltpu.bitcast(mant, jnp.float32) - 1.0
    u = jnp.maximum(_TINY, u0 + _TINY)
    return -jnp.log(-jnp.log(u))


def _rowroll(a, j):
    if j % a.shape[0] == 0:
        return a
    return jnp.concatenate([a[j:], a[:j]], axis=0)


def _fshift(a, k, mask):
    kl, kr = k % _LANES, k // _LANES
    if kl == 0:
        return _rowroll(a, kr)
    b = jnp.roll(a, -kl, axis=1)
    return jnp.where(mask, _rowroll(b, kr), _rowroll(b, kr + 1))


def _bshift(a, k, mask):
    kl, kr = k % _LANES, k // _LANES
    if kl == 0:
        return _rowroll(a, -kr)
    b = jnp.roll(a, kl, axis=1)
    return jnp.where(mask, _rowroll(b, -kr), _rowroll(b, -(kr + 1)))


def _pool_kernel(l_ref, valid_ref, norm_ref, x_ref, o_ref, *, W, L, imgs):
    R = l_ref.shape[0]
    l_arr = l_ref[...]
    valid = valid_ref[...]
    norm = norm_ref[...]
    lane = lax.broadcasted_iota(jnp.int32, (R, _LANES), 1)
    kW, kW1 = W % _LANES, (W + 1) % _LANES
    m_f1 = lane < (_LANES - 1)
    m_fw = lane < (_LANES - kW)
    m_fw1 = lane < (_LANES - kW1)
    m_b1 = lane >= 1
    m_bw = lane >= kW
    m_bw1 = lane >= kW1
    g0 = pl.program_id(0) * imgs

    for i in range(imgs):
        xf = x_ref[i]
        xa = _fshift(xf, 1, m_f1)
        xb = _fshift(xf, W, m_fw)
        xc = _fshift(xf, W + 1, m_fw1)
        rec = 1.0 / ((((xf + xa) + xb) + xc) + _EPS)
        base = (g0 + i) * (4 * L) + l_arr

        best = None
        idx = None
        for q, f in enumerate((xf, xa, xb, xc)):
            g = _gumbel((base + q * L).astype(jnp.uint32))
            v = jnp.log(jnp.maximum(f * rec, 1e-30)) + g
            if q == 0:
                best, idx = v, jnp.zeros_like(l_arr)
            else:
                take = v > best
                idx = jnp.where(take, q, idx)
                best = jnp.maximum(best, v)

        c0 = jnp.where(idx == 0, valid, 0.0)
        c1 = jnp.where(idx == 1, valid, 0.0)
        c2 = jnp.where(idx == 2, valid, 0.0)
        c3 = jnp.where(idx == 3, valid, 0.0)
        m = (c0 + _bshift(c1, 1, m_b1)
             + _bshift(c2 + _bshift(c3, 1, m_b1), W, m_bw))
        o_ref[i] = (xf * m) * norm


def _pixel_constants(H, W, R):
    i = np.arange(H * W)
    h, w = i // W, i % W
    l_arr = (h * (W - 1) + w).astype(np.int32).reshape(R, _LANES)
    valid = ((h < H - 1) & (w < W - 1)).astype(np.float32).reshape(R, _LANES)
    inv_r = np.where((h == 0) | (h == H - 1), 1.0, 0.5)
    inv_c = np.where((w == 0) | (w == W - 1), 1.0, 0.5)
    norm = (inv_r * inv_c).astype(np.float32).reshape(R, _LANES)
    return l_arr, valid, norm


@jax.jit
def kernel(x):
    B, C, H, W = x.shape
    bc = B * C
    L = (H - 1) * (W - 1)
    R = (H * W) // _LANES
    imgs = _IMGS if bc % _IMGS == 0 else 1
    l_arr, valid, norm = _pixel_constants(H, W, R)
    xr = x.reshape(bc, R, _LANES)
    body = functools.partial(_pool_kernel, W=W, L=L, imgs=imgs)
    const_spec = pl.BlockSpec((R, _LANES), lambda b: (0, 0))
    out = pl.pallas_call(
        body,
        grid=(bc // imgs,),
        in_specs=[const_spec, const_spec, const_spec,
                  pl.BlockSpec((imgs, R, _LANES), lambda b: (b, 0, 0))],
        out_specs=pl.BlockSpec((imgs, R, _LANES), lambda b: (b, 0, 0)),
        out_shape=jax.ShapeDtypeStruct((bc, R, _LANES), x.dtype),
        compiler_params=pltpu.CompilerParams(
            dimension_semantics=("arbitrary",)),
    )(jnp.asarray(l_arr), jnp.asarray(valid), jnp.asarray(norm), xr)
    return out.reshape(B, C, H, W)


# speedup vs baseline: 1.0628x; 1.0628x over previous
import functools

import jax
import jax.numpy as jnp
from jax import lax
from jax.experimental import pallas as pl
from jax.experimental.pallas import tpu as pltpu

_TINY = 1.1754943508222875e-38
_KS1 = 42
_KS2 = 0x1BD11BF0
_ROT = ((13, 15, 26, 6), (17, 29, 16, 24))
_IMGS = 4


def _threefry_bits(n):
    ks = (0, _KS1, _KS2)
    x0 = jnp.zeros_like(n)
    x1 = n + jnp.uint32(_KS1)
    for i in range(5):
        for r in _ROT[i % 2]:
            x0 = x0 + x1
            x1 = (x1 << r) | (x1 >> (32 - r))
            x1 = x1 ^ x0
        x0 = x0 + jnp.uint32(ks[(i + 1) % 3])
        x1 = x1 + jnp.uint32(ks[(i + 2) % 3] + (i + 1))
    return x0 ^ x1


def _neg_log2_u(n):
    bits = _threefry_bits(n)
    mant = (bits >> 9) | jnp.uint32(0x3F800000)
    u0 = pltpu.bitcast(mant, jnp.float32) - 1.0
    u = jnp.maximum(_TINY, u0 + _TINY)
    return -jnp.log2(u)


def _shift_m1(a, axis):
    n = a.shape[axis]
    return jnp.concatenate(
        [lax.slice_in_dim(a, 1, n, axis=axis),
         lax.slice_in_dim(a, 0, 1, axis=axis)], axis=axis)


def _shift_p1(a, axis):
    n = a.shape[axis]
    return jnp.concatenate(
        [lax.slice_in_dim(a, n - 1, n, axis=axis),
         lax.slice_in_dim(a, 0, n - 1, axis=axis)], axis=axis)


def _pool_kernel(x_ref, o_ref, *, Hout, Wout, L, imgs):
    H, W = x_ref.shape[1], x_ref.shape[2]
    hh = lax.broadcasted_iota(jnp.int32, (H, W), 0)
    ww = lax.broadcasted_iota(jnp.int32, (H, W), 1)
    l = hh * Wout + ww
    valid = ((hh < Hout) & (ww < Wout)).astype(jnp.float32)
    inv_r = jnp.where((hh == 0) | (hh == H - 1), 1.0, 0.5)
    inv_c = jnp.where((ww == 0) | (ww == W - 1), 1.0, 0.5)
    norm = inv_r * inv_c
    g0 = pl.program_id(0) * imgs

    for i in range(imgs):
        xv = x_ref[i]
        x01 = _shift_m1(xv, 1)
        x10 = _shift_m1(xv, 0)
        x11 = _shift_m1(x10, 1)
        base = (g0 + i) * (4 * L) + l

        best = None
        idx = None
        for q, f in enumerate((xv, x01, x10, x11)):
            d = _neg_log2_u((base + q * L).astype(jnp.uint32))
            v = jnp.log2(jnp.maximum(f, 1e-30)) - jnp.log2(d)
            if q == 0:
                best, idx = v, jnp.zeros_like(l)
            else:
                take = v > best
                idx = jnp.where(take, q, idx)
                best = jnp.maximum(best, v)

        c0 = jnp.where(idx == 0, valid, 0.0)
        c1 = jnp.where(idx == 1, valid, 0.0)
        c2 = jnp.where(idx == 2, valid, 0.0)
        c3 = jnp.where(idx == 3, valid, 0.0)
        m = c0 + _shift_p1(c1, 1) + _shift_p1(c2 + _shift_p1(c3, 1), 0)
        o_ref[i] = (xv * m) * norm


@jax.jit
def kernel(x):
    B, C, H, W = x.shape
    bc = B * C
    Hout, Wout = H - 1, W - 1
    L = Hout * Wout
    imgs = _IMGS if bc % _IMGS == 0 else 1
    xr = x.reshape(bc, H, W)
    body = functools.partial(_pool_kernel, Hout=Hout, Wout=Wout, L=L,
                             imgs=imgs)
    out = pl.pallas_call(
        body,
        grid=(bc // imgs,),
        in_specs=[pl.BlockSpec((imgs, H, W), lambda b: (b, 0, 0))],
        out_specs=pl.BlockSpec((imgs, H, W), lambda b: (b, 0, 0)),
        out_shape=jax.ShapeDtypeStruct((bc, H, W), x.dtype),
        compiler_params=pltpu.CompilerParams(
            dimension_semantics=("arbitrary",)),
    )(xr)
    return out.reshape(B, C, H, W)
